# baseline (device time: 9936 ns/iter reference)
import jax
import jax.numpy as jnp
from jax import lax
from jax.experimental import pallas as pl
from jax.experimental.pallas import tpu as pltpu

N_CHUNKS = 16


def kernel(x):
    m, n = x.shape
    ck = m // N_CHUNKS

    def body(x_ref, out_ref, part_ref, comm_ref, send_sems, recv_sems):
        my_x = lax.axis_index("x")
        my_y = lax.axis_index("y")
        y_nbr = (my_x, 1 - my_y)
        x_nbr = (1 - my_x, my_y)
        h = N_CHUNKS // 2
        p1_dev = {c: (y_nbr if c < h else x_nbr) for c in range(N_CHUNKS)}
        p2_dev = {c: (x_nbr if c < h else y_nbr) for c in range(N_CHUNKS)}
        arrival = [c for pair in zip(range(h), range(h, N_CHUNKS)) for c in pair]

        barrier_sem = pltpu.get_barrier_semaphore()
        for nbr in (y_nbr, x_nbr):
            pl.semaphore_signal(
                barrier_sem, inc=1,
                device_id=nbr, device_id_type=pl.DeviceIdType.MESH,
            )
        pl.semaphore_wait(barrier_sem, 2)

        p1 = [None] * N_CHUNKS
        for c in arrival:
            rdma = pltpu.make_async_remote_copy(
                src_ref=x_ref.at[pl.ds(c * ck, ck)],
                dst_ref=comm_ref.at[c],
                send_sem=send_sems.at[c],
                recv_sem=recv_sems.at[c],
                device_id=p1_dev[c],
                device_id_type=pl.DeviceIdType.MESH,
            )
            rdma.start()
            p1[c] = rdma

        p2 = [None] * N_CHUNKS
        for c in arrival:
            p1[c].wait_recv()
            part_ref[pl.ds(c * ck, ck), :] = (
                x_ref[pl.ds(c * ck, ck), :] + comm_ref[c, :, :]
            )
            rdma = pltpu.make_async_remote_copy(
                src_ref=part_ref.at[pl.ds(c * ck, ck)],
                dst_ref=comm_ref.at[N_CHUNKS + c],
                send_sem=send_sems.at[N_CHUNKS + c],
                recv_sem=recv_sems.at[N_CHUNKS + c],
                device_id=p2_dev[c],
                device_id_type=pl.DeviceIdType.MESH,
            )
            rdma.start()
            p2[c] = rdma

        for c in arrival:
            p2[c].wait_recv()
            out_ref[pl.ds(c * ck, ck), :] = (
                part_ref[pl.ds(c * ck, ck), :] + comm_ref[N_CHUNKS + c, :, :]
            )

        for rdma in p1 + p2:
            rdma.wait_send()

    return pl.pallas_call(
        body,
        out_shape=jax.ShapeDtypeStruct((m, n), x.dtype),
        in_specs=[pl.BlockSpec(memory_space=pltpu.VMEM)],
        out_specs=pl.BlockSpec(memory_space=pltpu.VMEM),
        scratch_shapes=[
            pltpu.VMEM((m, n), x.dtype),
            pltpu.VMEM((2 * N_CHUNKS, ck, n), x.dtype),
            pltpu.SemaphoreType.DMA((2 * N_CHUNKS,)),
            pltpu.SemaphoreType.DMA((2 * N_CHUNKS,)),
        ],
        compiler_params=pltpu.CompilerParams(collective_id=0),
    )(x)


# device time: 9786 ns/iter; 1.0153x vs baseline; 1.0153x over previous
import jax
import jax.numpy as jnp
from jax import lax
from jax.experimental import pallas as pl
from jax.experimental.pallas import tpu as pltpu

N_CHUNKS = 8


def kernel(x):
    m, n = x.shape
    ck = m // N_CHUNKS

    def body(x_ref, out_ref, part_ref, comm_ref, send_sems, recv_sems):
        my_x = lax.axis_index("x")
        my_y = lax.axis_index("y")
        y_nbr = (my_x, 1 - my_y)
        x_nbr = (1 - my_x, my_y)
        h = N_CHUNKS // 2
        p1_dev = {c: (y_nbr if c < h else x_nbr) for c in range(N_CHUNKS)}
        p2_dev = {c: (x_nbr if c < h else y_nbr) for c in range(N_CHUNKS)}
        arrival = [c for pair in zip(range(h), range(h, N_CHUNKS)) for c in pair]

        barrier_sem = pltpu.get_barrier_semaphore()
        for nbr in (y_nbr, x_nbr):
            pl.semaphore_signal(
                barrier_sem, inc=1,
                device_id=nbr, device_id_type=pl.DeviceIdType.MESH,
            )
        pl.semaphore_wait(barrier_sem, 2)

        p1 = [None] * N_CHUNKS
        for c in arrival:
            rdma = pltpu.make_async_remote_copy(
                src_ref=x_ref.at[pl.ds(c * ck, ck)],
                dst_ref=comm_ref.at[c],
                send_sem=send_sems.at[c],
                recv_sem=recv_sems.at[c],
                device_id=p1_dev[c],
                device_id_type=pl.DeviceIdType.MESH,
            )
            rdma.start()
            p1[c] = rdma

        p2 = [None] * N_CHUNKS
        for c in arrival:
            p1[c].wait_recv()
            part_ref[pl.ds(c * ck, ck), :] = (
                x_ref[pl.ds(c * ck, ck), :] + comm_ref[c, :, :]
            )
            rdma = pltpu.make_async_remote_copy(
                src_ref=part_ref.at[pl.ds(c * ck, ck)],
                dst_ref=comm_ref.at[N_CHUNKS + c],
                send_sem=send_sems.at[N_CHUNKS + c],
                recv_sem=recv_sems.at[N_CHUNKS + c],
                device_id=p2_dev[c],
                device_id_type=pl.DeviceIdType.MESH,
            )
            rdma.start()
            p2[c] = rdma

        for c in arrival:
            p2[c].wait_recv()
            out_ref[pl.ds(c * ck, ck), :] = (
                part_ref[pl.ds(c * ck, ck), :] + comm_ref[N_CHUNKS + c, :, :]
            )

        for rdma in p1 + p2:
            rdma.wait_send()

    return pl.pallas_call(
        body,
        out_shape=jax.ShapeDtypeStruct((m, n), x.dtype),
        in_specs=[pl.BlockSpec(memory_space=pltpu.VMEM)],
        out_specs=pl.BlockSpec(memory_space=pltpu.VMEM),
        scratch_shapes=[
            pltpu.VMEM((m, n), x.dtype),
            pltpu.VMEM((2 * N_CHUNKS, ck, n), x.dtype),
            pltpu.SemaphoreType.DMA((2 * N_CHUNKS,)),
            pltpu.SemaphoreType.DMA((2 * N_CHUNKS,)),
        ],
        compiler_params=pltpu.CompilerParams(collective_id=0),
    )(x)
